# Initial kernel scaffold; baseline (speedup 1.0000x reference)
import jax
import jax.numpy as jnp
from jax.experimental import pallas as pl


def kernel(Z, element_embedding, W, electron_config):
    raise NotImplementedError("write your pallas kernel here")


# speedup vs baseline: 1.3625x; 1.3625x over previous
import jax
import jax.numpy as jnp
from jax import lax
from jax.experimental import pallas as pl
from jax.experimental.pallas import tpu as pltpu
from jax.experimental.pallas import tpu_sc as plsc

N_ATOMS = 100000
D = 128
ZMAX = 87

NC = 2
NS = 16
NW = NC * NS

CHUNK = 128
N_CHUNKS = (N_ATOMS + CHUNK - 1) // CHUNK
MAX_PER_WORKER = (N_CHUNKS + NW - 1) // NW


def _table_body(emb_ref, ec_ref, w_ref, out_ref):
    out_ref[...] = emb_ref[...] + lax.dot_general(
        ec_ref[...], w_ref[...],
        dimension_numbers=(((1,), (1,)), ((), ())),
        preferred_element_type=jnp.float32,
    )


def _build_table(element_embedding, W, electron_config):
    return pl.pallas_call(
        _table_body,
        out_shape=jax.ShapeDtypeStruct((ZMAX, D), jnp.float32),
    )(element_embedding, electron_config, W)


def _gather_body(table_hbm, z_hbm, out_hbm, idx_v, rows_v, sem):
    wid = lax.axis_index("s") * NC + lax.axis_index("c")
    for k in range(MAX_PER_WORKER):
        c = wid + k * NW

        @pl.when(c < N_CHUNKS)
        def _():
            base = jnp.minimum(c * CHUNK, N_ATOMS - CHUNK)
            pltpu.sync_copy(z_hbm.at[pl.ds(base, CHUNK)], idx_v)
            pltpu.async_copy(table_hbm.at[idx_v], rows_v, sem).wait()
            pltpu.sync_copy(rows_v, out_hbm.at[pl.ds(base, CHUNK)])


_gather = pl.kernel(
    _gather_body,
    out_type=jax.ShapeDtypeStruct((N_ATOMS, D), jnp.float32),
    mesh=plsc.VectorSubcoreMesh(core_axis_name="c", subcore_axis_name="s"),
    scratch_types=[
        pltpu.VMEM((CHUNK,), jnp.int32),
        pltpu.VMEM((CHUNK, D), jnp.float32),
        pltpu.SemaphoreType.DMA,
    ],
)


def kernel(Z, element_embedding, W, electron_config):
    table = _build_table(element_embedding, W, electron_config)
    return _gather(table, Z.astype(jnp.int32))
